# baseline (speedup 1.0000x reference)
import functools

import jax
import jax.numpy as jnp
from jax import lax
from jax.experimental import pallas as pl
from jax.experimental.pallas import tpu as pltpu
from jax.experimental.pallas import tpu_sc as plsc

N = 10000
D = 128
OUT = 64
E = 320000
K_EGO = 4
E_EGO = 80000

NC = 2
NS = 16
NW = NC * NS
CH = 64
CHT = 316
EP16 = NS * CH * CHT
CPW = 160
EPC = NW * CH * CPW
DH = D // NC
NPAD = NW * 640

_f32 = jnp.float32
_i32 = jnp.int32

_SC_PARAMS = pltpu.CompilerParams(needs_layout_passes=False,
                                  use_tc_tiling_on_sc=False)


def _sc_phase_a(edata, cdata, x):
    mesh = plsc.VectorSubcoreMesh(core_axis_name="c", subcore_axis_name="s",
                                  num_cores=NC, num_subcores=NS)

    @functools.partial(
        pl.kernel,
        out_type=(
            jax.ShapeDtypeStruct((NC, NPAD, DH), _f32),
            jax.ShapeDtypeStruct((NC, NPAD), _f32),
        ),
        mesh=mesh,
        compiler_params=_SC_PARAMS,
        scratch_types=[
            pltpu.VMEM_SHARED((NPAD, DH), _f32),
            pltpu.VMEM_SHARED((NPAD,), _f32),
            pltpu.VMEM((3, CH), _i32),
            pltpu.VMEM((3, CH), _i32),
            pltpu.VMEM((3, CH), _i32),
            pltpu.VMEM((3, CH), _i32),
            pltpu.VMEM((CH, DH), _f32),
            pltpu.VMEM((CH, DH), _f32),
            pltpu.VMEM((CH, DH), _f32),
            pltpu.VMEM((CH, DH), _f32),
            pltpu.VMEM((CH,), _i32),
            pltpu.VMEM((CH,), _i32),
            pltpu.VMEM((CH,), _f32),
            pltpu.VMEM((16, DH), _f32),
            pltpu.SemaphoreType.DMA,
            pltpu.SemaphoreType.DMA,
            pltpu.SemaphoreType.DMA,
            pltpu.SemaphoreType.DMA,
            pltpu.SemaphoreType.DMA,
            pltpu.SemaphoreType.DMA,
            pltpu.SemaphoreType.DMA,
            pltpu.SemaphoreType.DMA,
            pltpu.SemaphoreType.DMA,
            pltpu.SemaphoreType.DMA,
            pltpu.SemaphoreType.DMA,
            pltpu.SemaphoreType.DMA,
            pltpu.SemaphoreType.DMA,
            pltpu.SemaphoreType.DMA,
            pltpu.SemaphoreType.DMA,
        ],
    )
    def k(ed_h, cd_h, x_h, acc_out, deg_out,
          acc_sh, deg_sh, e0, e1, e2, e3, r0, r1, r2, r3, c0, c1,
          ones_v, zb_v, es0, es1, es2, es3, gs0, gs1, gs2, gs3,
          ss0, ss1, ss2, ss3, cs0, cs1, zsem):
        cid = lax.axis_index("c")
        sid = lax.axis_index("s")
        wid = cid * NS + sid
        lane = lax.iota(_i32, 16)
        ebufs = (e0, e1, e2, e3)
        esems = (es0, es1, es2, es3)
        rbufs = (r0, r1, r2, r3)
        gsems = (gs0, gs1, gs2, gs3)
        ssems = (ss0, ss1, ss2, ss3)
        cbufs = (c0, c1)
        csems = (cs0, cs1)

        z = jnp.zeros((16,), _f32)
        one = jnp.ones((16,), _f32)
        for j in range(16):
            for q in range(DH // 16):
                zb_v[j, pl.ds(q * 16, 16)] = z

        for g in range(CH // 16):
            ones_v[pl.ds(g * 16, 16)] = z

        def zacc(i, carry):
            pltpu.async_copy(zb_v, acc_sh.at[pl.ds(sid * 640 + i * 16, 16)],
                             zsem)
            return carry
        lax.fori_loop(0, 40, zacc, 0)
        for j in range(10):
            pltpu.async_copy(ones_v,
                             deg_sh.at[pl.ds(sid * 640 + j * CH, CH)], zsem)

        def zaccw(i, carry):
            pltpu.make_async_copy(
                zb_v, acc_sh.at[pl.ds(sid * 640 + i * 16, 16)], zsem).wait()
            return carry
        lax.fori_loop(0, 40, zaccw, 0)
        for j in range(10):
            pltpu.make_async_copy(
                ones_v, deg_sh.at[pl.ds(sid * 640 + j * CH, CH)],
                zsem).wait()
        for g in range(CH // 16):
            ones_v[pl.ds(g * 16, 16)] = one
        plsc.subcore_barrier()

        def stage(ci, b4):
            pltpu.async_copy(ed_h.at[sid * CHT + ci], ebufs[b4], esems[b4])

        def wait_stage(ci, b4):
            pltpu.make_async_copy(ed_h.at[sid * CHT + ci], ebufs[b4],
                                  esems[b4]).wait()

        def gather(b4):
            pltpu.async_copy(x_h.at[cid].at[ebufs[b4].at[0]],
                             rbufs[b4], gsems[b4])

        def wait_gather(b4):
            pltpu.make_async_copy(x_h.at[cid].at[ebufs[b4].at[0]],
                                  rbufs[b4], gsems[b4]).wait()

        def scatter(b4):
            pltpu.async_copy(rbufs[b4], acc_sh.at[ebufs[b4].at[1]],
                             ssems[b4], add=True)

        def wait_scatter(b4):
            pltpu.make_async_copy(rbufs[b4], acc_sh.at[ebufs[b4].at[1]],
                                  ssems[b4]).wait()

        def scale(b4):
            rows = rbufs[b4]
            eb = ebufs[b4]
            for g in range(CH // 16):
                vals16 = plsc.bitcast(eb[2, pl.ds(g * 16, 16)], _f32)
                for e16 in range(16):
                    e = g * 16 + e16
                    eidx = jnp.full((16,), e, _i32)
                    vb = vals16.at[jnp.full((16,), e16, _i32)].get(
                        mode="promise_in_bounds")
                    for q in range(DH // 16):
                        lidx = q * 16 + lane
                        r = plsc.load_gather(rows, [eidx, lidx])
                        plsc.store_scatter(rows, [eidx, lidx], r * vb)

        stage(0, 0)
        stage(1, 1)
        wait_stage(0, 0)
        gather(0)

        def section(ci, b4):
            @pl.when(ci + 1 < CHT)
            def _():
                wait_stage(ci + 1, (b4 + 1) % 4)
                gather((b4 + 1) % 4)

            @pl.when(ci + 2 < CHT)
            def _():
                @pl.when(ci >= 2)
                def _():
                    wait_scatter((b4 + 2) % 4)
                stage(ci + 2, (b4 + 2) % 4)
            wait_gather(b4)
            scale(b4)
            scatter(b4)

        def quad(q4, carry):
            for s in range(4):
                section(q4 * 4 + s, s)
            return carry
        lax.fori_loop(0, CHT // 4, quad, 0)
        for b in range(4):
            wait_scatter(b)

        pltpu.async_copy(cd_h.at[wid * CPW, 1], c0, cs0)

        def dchunk(ci2, carry):
            for b in range(2):
                ci = ci2 * 2 + b

                @pl.when(ci + 1 < CPW)
                def _():
                    pltpu.async_copy(cd_h.at[wid * CPW + ci + 1, 1],
                                     cbufs[1 - b], csems[1 - b])
                pltpu.make_async_copy(cd_h.at[wid * CPW + ci, 1],
                                      cbufs[b], csems[b]).wait()
                pltpu.sync_copy(ones_v, deg_sh.at[cbufs[b]], add=True)
            return carry
        lax.fori_loop(0, CPW // 2, dchunk, 0)

        plsc.subcore_barrier()

        pltpu.sync_copy(acc_sh.at[pl.ds(sid * 640, 640)],
                        acc_out.at[cid, pl.ds(sid * 640, 640)])
        pltpu.sync_copy(deg_sh.at[pl.ds(sid * 640, 640)],
                        deg_out.at[cid, pl.ds(sid * 640, 640)])

    return k(edata, cdata, x)


def _sc_phase_c(cdata, h2):
    mesh = plsc.VectorSubcoreMesh(core_axis_name="c", subcore_axis_name="s",
                                  num_cores=NC, num_subcores=NS)

    @functools.partial(
        pl.kernel,
        out_type=jax.ShapeDtypeStruct((NC, NPAD, OUT), _f32),
        mesh=mesh,
        compiler_params=_SC_PARAMS,
        scratch_types=[
            pltpu.VMEM_SHARED((NPAD, OUT), _f32),
            pltpu.VMEM((2, CH), _i32),
            pltpu.VMEM((2, CH), _i32),
            pltpu.VMEM((2, CH), _i32),
            pltpu.VMEM((2, CH), _i32),
            pltpu.VMEM((CH, OUT), _f32),
            pltpu.VMEM((CH, OUT), _f32),
            pltpu.VMEM((CH, OUT), _f32),
            pltpu.VMEM((CH, OUT), _f32),
            pltpu.VMEM((16, OUT), _f32),
            pltpu.SemaphoreType.DMA,
            pltpu.SemaphoreType.DMA,
            pltpu.SemaphoreType.DMA,
            pltpu.SemaphoreType.DMA,
            pltpu.SemaphoreType.DMA,
            pltpu.SemaphoreType.DMA,
            pltpu.SemaphoreType.DMA,
            pltpu.SemaphoreType.DMA,
            pltpu.SemaphoreType.DMA,
            pltpu.SemaphoreType.DMA,
            pltpu.SemaphoreType.DMA,
            pltpu.SemaphoreType.DMA,
            pltpu.SemaphoreType.DMA,
        ],
    )
    def k(cd_h, h2_h, agg_out, agg_sh, e0, e1, e2, e3, r0, r1, r2, r3,
          zb_v, es0, es1, es2, es3, gs0, gs1, gs2, gs3, ss0, ss1, ss2, ss3,
          zsem):
        cid = lax.axis_index("c")
        sid = lax.axis_index("s")
        wid = cid * NS + sid
        ebufs = (e0, e1, e2, e3)
        esems = (es0, es1, es2, es3)
        rbufs = (r0, r1, r2, r3)
        gsems = (gs0, gs1, gs2, gs3)
        ssems = (ss0, ss1, ss2, ss3)

        z = jnp.zeros((16,), _f32)
        for j in range(16):
            for q in range(OUT // 16):
                zb_v[j, pl.ds(q * 16, 16)] = z

        def zacc(i, carry):
            pltpu.async_copy(zb_v, agg_sh.at[pl.ds(sid * 640 + i * 16, 16)],
                             zsem)
            return carry
        lax.fori_loop(0, 40, zacc, 0)

        def zaccw(i, carry):
            pltpu.make_async_copy(
                zb_v, agg_sh.at[pl.ds(sid * 640 + i * 16, 16)], zsem).wait()
            return carry
        lax.fori_loop(0, 40, zaccw, 0)
        plsc.subcore_barrier()

        def stage(ci, b4):
            pltpu.async_copy(cd_h.at[wid * CPW + ci], ebufs[b4], esems[b4])

        def wait_stage(ci, b4):
            pltpu.make_async_copy(cd_h.at[wid * CPW + ci], ebufs[b4],
                                  esems[b4]).wait()

        def gather(b4):
            pltpu.async_copy(h2_h.at[ebufs[b4].at[0]], rbufs[b4], gsems[b4])

        def wait_gather(b4):
            pltpu.make_async_copy(h2_h.at[ebufs[b4].at[0]], rbufs[b4],
                                  gsems[b4]).wait()

        def scatter(b4):
            pltpu.async_copy(rbufs[b4], agg_sh.at[ebufs[b4].at[1]],
                             ssems[b4], add=True)

        def wait_scatter(b4):
            pltpu.make_async_copy(rbufs[b4], agg_sh.at[ebufs[b4].at[1]],
                                  ssems[b4]).wait()

        stage(0, 0)
        stage(1, 1)
        wait_stage(0, 0)
        gather(0)

        def section(ci, b4):
            @pl.when(ci + 1 < CPW)
            def _():
                wait_stage(ci + 1, (b4 + 1) % 4)
                gather((b4 + 1) % 4)

            @pl.when(ci + 2 < CPW)
            def _():
                @pl.when(ci >= 2)
                def _():
                    wait_scatter((b4 + 2) % 4)
                stage(ci + 2, (b4 + 2) % 4)
            wait_gather(b4)
            scatter(b4)

        def quad(q4, carry):
            for s in range(4):
                section(q4 * 4 + s, s)
            return carry
        lax.fori_loop(0, CPW // 4, quad, 0)
        for b in range(4):
            wait_scatter(b)

        plsc.subcore_barrier()
        pltpu.sync_copy(agg_sh.at[pl.ds(sid * 640, 640)],
                        agg_out.at[cid, pl.ds(sid * 640, 640)])

    return k(cdata, h2)


_RB = 1000


def _tc_b(acc_pair, deg_pair, norm_degrees, W_ego, b_ego, W_gcn):
    def body(acc_ref, deg_ref, nd_ref, we_ref, be_ref, wg_ref, h2_ref):
        ego = jnp.concatenate([acc_ref[0], acc_ref[1]], axis=1) * nd_ref[...]
        h1 = jnp.maximum(
            jnp.dot(ego, we_ref[...], preferred_element_type=_f32)
            + be_ref[...], 0.0)
        deg = deg_ref[0] + deg_ref[1] + 1.0
        dis = lax.rsqrt(jnp.maximum(deg, 1e-12))
        h2_ref[...] = dis * jnp.dot(h1, wg_ref[...],
                                    preferred_element_type=_f32)

    return pl.pallas_call(
        body,
        grid=(N // _RB,),
        in_specs=[
            pl.BlockSpec((NC, _RB, DH), lambda i: (0, i, 0)),
            pl.BlockSpec((NC, _RB, 1), lambda i: (0, i, 0)),
            pl.BlockSpec((_RB, 1), lambda i: (i, 0)),
            pl.BlockSpec((D, D), lambda i: (0, 0)),
            pl.BlockSpec((1, D), lambda i: (0, 0)),
            pl.BlockSpec((D, OUT), lambda i: (0, 0)),
        ],
        out_specs=pl.BlockSpec((_RB, OUT), lambda i: (i, 0)),
        out_shape=jax.ShapeDtypeStruct((N, OUT), _f32),
    )(acc_pair, deg_pair, norm_degrees, W_ego, b_ego.reshape(1, D), W_gcn)


def _tc_d(agg_pair, h2, deg_pair, b_gcn):
    def body(agg_ref, h2_ref, deg_ref, bg_ref, out_ref):
        deg = deg_ref[0] + deg_ref[1] + 1.0
        dis = lax.rsqrt(jnp.maximum(deg, 1e-12))
        g = dis * (agg_ref[0] + agg_ref[1] + h2_ref[...]) + bg_ref[...]
        m = jnp.max(g, axis=1, keepdims=True)
        lse = m + jnp.log(jnp.sum(jnp.exp(g - m), axis=1, keepdims=True))
        out_ref[...] = g - lse

    return pl.pallas_call(
        body,
        grid=(N // _RB,),
        in_specs=[
            pl.BlockSpec((NC, _RB, OUT), lambda i: (0, i, 0)),
            pl.BlockSpec((_RB, OUT), lambda i: (i, 0)),
            pl.BlockSpec((NC, _RB, 1), lambda i: (0, i, 0)),
            pl.BlockSpec((1, OUT), lambda i: (0, 0)),
        ],
        out_specs=pl.BlockSpec((_RB, OUT), lambda i: (i, 0)),
        out_shape=jax.ShapeDtypeStruct((N, OUT), _f32),
    )(agg_pair, h2, deg_pair, b_gcn.reshape(1, OUT))


def kernel(x_in, edge_index_in, ego_ind, ego_val, norm_degrees, W_ego,
           b_ego, W_gcn, b_gcn):
    src = jnp.pad(ego_ind[:, 1, :].reshape(-1), (0, EP16 - E)).reshape(
        NS * CHT, CH)
    dst = jnp.pad(ego_ind[:, 0, :].reshape(-1), (0, EP16 - E)).reshape(
        NS * CHT, CH)
    val = lax.bitcast_convert_type(
        jnp.pad(ego_val.reshape(-1), (0, EP16 - E)), _i32).reshape(
        NS * CHT, CH)
    edata = jnp.stack([src, dst, val], axis=1)
    row = jnp.pad(edge_index_in[0], (0, EPC - E)).reshape(NW * CPW, CH)
    col = jnp.pad(edge_index_in[1], (0, EPC - E),
                  constant_values=N).reshape(NW * CPW, CH)
    cdata = jnp.stack([row, col], axis=1)
    x_halves = jnp.stack([x_in[:, :DH], x_in[:, DH:]])

    acc_full, deg_full = _sc_phase_a(edata, cdata, x_halves)
    deg_all = deg_full.reshape(NC, NPAD, 1)
    h2 = _tc_b(acc_full, deg_all, norm_degrees, W_ego, b_ego, W_gcn)
    agg_full = _sc_phase_c(cdata, h2)
    return _tc_d(agg_full, h2, deg_all, b_gcn)


# speedup vs baseline: 1.3393x; 1.3393x over previous
import functools

import jax
import jax.numpy as jnp
from jax import lax
from jax.experimental import pallas as pl
from jax.experimental.pallas import tpu as pltpu
from jax.experimental.pallas import tpu_sc as plsc

N = 10000
D = 128
OUT = 64
E = 320000
K_EGO = 4
E_EGO = 80000

NC = 2
NS = 16
NW = NC * NS
CH = 64
CHT = 316
EP16 = NS * CH * CHT
CPW = 160
EPC = NW * CH * CPW
DH = D // NC
NPAD = NW * 640

_f32 = jnp.float32
_i32 = jnp.int32

_SC_PARAMS = pltpu.CompilerParams(needs_layout_passes=False,
                                  use_tc_tiling_on_sc=False)


def _sc_phase_a(edata, cdata, x):
    mesh = plsc.VectorSubcoreMesh(core_axis_name="c", subcore_axis_name="s",
                                  num_cores=NC, num_subcores=NS)

    @functools.partial(
        pl.kernel,
        out_type=(
            jax.ShapeDtypeStruct((NC, NPAD, DH), _f32),
            jax.ShapeDtypeStruct((NC, NPAD), _f32),
        ),
        mesh=mesh,
        compiler_params=_SC_PARAMS,
        scratch_types=[
            pltpu.VMEM_SHARED((NPAD, DH), _f32),
            pltpu.VMEM_SHARED((NPAD,), _f32),
            pltpu.VMEM((3, CH), _i32),
            pltpu.VMEM((3, CH), _i32),
            pltpu.VMEM((3, CH), _i32),
            pltpu.VMEM((3, CH), _i32),
            pltpu.VMEM((CH, DH), _f32),
            pltpu.VMEM((CH, DH), _f32),
            pltpu.VMEM((CH, DH), _f32),
            pltpu.VMEM((CH, DH), _f32),
            pltpu.VMEM((CH,), _i32),
            pltpu.VMEM((CH,), _i32),
            pltpu.VMEM((CH,), _f32),
            pltpu.VMEM((16, DH), _f32),
            pltpu.SemaphoreType.DMA,
            pltpu.SemaphoreType.DMA,
            pltpu.SemaphoreType.DMA,
            pltpu.SemaphoreType.DMA,
            pltpu.SemaphoreType.DMA,
            pltpu.SemaphoreType.DMA,
            pltpu.SemaphoreType.DMA,
            pltpu.SemaphoreType.DMA,
            pltpu.SemaphoreType.DMA,
            pltpu.SemaphoreType.DMA,
            pltpu.SemaphoreType.DMA,
            pltpu.SemaphoreType.DMA,
            pltpu.SemaphoreType.DMA,
            pltpu.SemaphoreType.DMA,
            pltpu.SemaphoreType.DMA,
        ],
    )
    def k(ed_h, cd_h, x_h, acc_out, deg_out,
          acc_sh, deg_sh, e0, e1, e2, e3, r0, r1, r2, r3, c0, c1,
          ones_v, zb_v, es0, es1, es2, es3, gs0, gs1, gs2, gs3,
          ss0, ss1, ss2, ss3, cs0, cs1, zsem):
        cid = lax.axis_index("c")
        sid = lax.axis_index("s")
        wid = cid * NS + sid
        lane = lax.iota(_i32, 16)
        ebufs = (e0, e1, e2, e3)
        esems = (es0, es1, es2, es3)
        rbufs = (r0, r1, r2, r3)
        gsems = (gs0, gs1, gs2, gs3)
        ssems = (ss0, ss1, ss2, ss3)
        cbufs = (c0, c1)
        csems = (cs0, cs1)

        z = jnp.zeros((16,), _f32)
        one = jnp.ones((16,), _f32)
        for j in range(16):
            for q in range(DH // 16):
                zb_v[j, pl.ds(q * 16, 16)] = z

        for g in range(CH // 16):
            ones_v[pl.ds(g * 16, 16)] = z

        def zacc(i, carry):
            pltpu.async_copy(zb_v, acc_sh.at[pl.ds(sid * 640 + i * 16, 16)],
                             zsem)
            return carry
        lax.fori_loop(0, 40, zacc, 0)
        for j in range(10):
            pltpu.async_copy(ones_v,
                             deg_sh.at[pl.ds(sid * 640 + j * CH, CH)], zsem)

        def zaccw(i, carry):
            pltpu.make_async_copy(
                zb_v, acc_sh.at[pl.ds(sid * 640 + i * 16, 16)], zsem).wait()
            return carry
        lax.fori_loop(0, 40, zaccw, 0)
        for j in range(10):
            pltpu.make_async_copy(
                ones_v, deg_sh.at[pl.ds(sid * 640 + j * CH, CH)],
                zsem).wait()
        for g in range(CH // 16):
            ones_v[pl.ds(g * 16, 16)] = one
        plsc.subcore_barrier()

        def stage(ci, b4):
            pltpu.async_copy(ed_h.at[sid * CHT + ci], ebufs[b4], esems[b4])

        def wait_stage(ci, b4):
            pltpu.make_async_copy(ed_h.at[sid * CHT + ci], ebufs[b4],
                                  esems[b4]).wait()

        def gather(b4):
            pltpu.async_copy(x_h.at[cid].at[ebufs[b4].at[0]],
                             rbufs[b4], gsems[b4])

        def wait_gather(b4):
            pltpu.make_async_copy(x_h.at[cid].at[ebufs[b4].at[0]],
                                  rbufs[b4], gsems[b4]).wait()

        def scatter(b4):
            pltpu.async_copy(rbufs[b4], acc_sh.at[ebufs[b4].at[1]],
                             ssems[b4], add=True)

        def wait_scatter(b4):
            pltpu.make_async_copy(rbufs[b4], acc_sh.at[ebufs[b4].at[1]],
                                  ssems[b4]).wait()

        def scale(b4):
            rows = rbufs[b4]
            eb = ebufs[b4]
            two = jnp.full((16,), 2, _i32)

            def grp(g, carry):
                for e8 in range(8):
                    e = g * 8 + e8
                    eidx = jnp.full((16,), e, _i32)
                    vb = plsc.bitcast(plsc.load_gather(eb, [two, eidx]),
                                      _f32)
                    for q in range(DH // 16):
                        lidx = q * 16 + lane
                        r = plsc.load_gather(rows, [eidx, lidx])
                        plsc.store_scatter(rows, [eidx, lidx], r * vb)
                return carry
            lax.fori_loop(0, CH // 8, grp, 0)

        stage(0, 0)
        stage(1, 1)
        wait_stage(0, 0)
        gather(0)

        def section(ci, b4):
            @pl.when(ci + 1 < CHT)
            def _():
                wait_stage(ci + 1, (b4 + 1) % 4)
                gather((b4 + 1) % 4)

            @pl.when(ci + 2 < CHT)
            def _():
                @pl.when(ci >= 2)
                def _():
                    wait_scatter((b4 + 2) % 4)
                stage(ci + 2, (b4 + 2) % 4)
            wait_gather(b4)
            scale(b4)
            scatter(b4)

        def quad(q4, carry):
            for s in range(4):
                section(q4 * 4 + s, s)
            return carry
        lax.fori_loop(0, CHT // 4, quad, 0)
        for b in range(4):
            wait_scatter(b)

        pltpu.async_copy(cd_h.at[wid * CPW, 1], c0, cs0)

        def dchunk(ci2, carry):
            for b in range(2):
                ci = ci2 * 2 + b

                @pl.when(ci + 1 < CPW)
                def _():
                    pltpu.async_copy(cd_h.at[wid * CPW + ci + 1, 1],
                                     cbufs[1 - b], csems[1 - b])
                pltpu.make_async_copy(cd_h.at[wid * CPW + ci, 1],
                                      cbufs[b], csems[b]).wait()
                pltpu.sync_copy(ones_v, deg_sh.at[cbufs[b]], add=True)
            return carry
        lax.fori_loop(0, CPW // 2, dchunk, 0)

        plsc.subcore_barrier()

        pltpu.sync_copy(acc_sh.at[pl.ds(sid * 640, 640)],
                        acc_out.at[cid, pl.ds(sid * 640, 640)])
        pltpu.sync_copy(deg_sh.at[pl.ds(sid * 640, 640)],
                        deg_out.at[cid, pl.ds(sid * 640, 640)])

    return k(edata, cdata, x)


def _sc_phase_c(cdata, h2):
    mesh = plsc.VectorSubcoreMesh(core_axis_name="c", subcore_axis_name="s",
                                  num_cores=NC, num_subcores=NS)

    @functools.partial(
        pl.kernel,
        out_type=jax.ShapeDtypeStruct((NC, NPAD, OUT), _f32),
        mesh=mesh,
        compiler_params=_SC_PARAMS,
        scratch_types=[
            pltpu.VMEM_SHARED((NPAD, OUT), _f32),
            pltpu.VMEM((2, CH), _i32),
            pltpu.VMEM((2, CH), _i32),
            pltpu.VMEM((2, CH), _i32),
            pltpu.VMEM((2, CH), _i32),
            pltpu.VMEM((CH, OUT), _f32),
            pltpu.VMEM((CH, OUT), _f32),
            pltpu.VMEM((CH, OUT), _f32),
            pltpu.VMEM((CH, OUT), _f32),
            pltpu.VMEM((16, OUT), _f32),
            pltpu.SemaphoreType.DMA,
            pltpu.SemaphoreType.DMA,
            pltpu.SemaphoreType.DMA,
            pltpu.SemaphoreType.DMA,
            pltpu.SemaphoreType.DMA,
            pltpu.SemaphoreType.DMA,
            pltpu.SemaphoreType.DMA,
            pltpu.SemaphoreType.DMA,
            pltpu.SemaphoreType.DMA,
            pltpu.SemaphoreType.DMA,
            pltpu.SemaphoreType.DMA,
            pltpu.SemaphoreType.DMA,
            pltpu.SemaphoreType.DMA,
        ],
    )
    def k(cd_h, h2_h, agg_out, agg_sh, e0, e1, e2, e3, r0, r1, r2, r3,
          zb_v, es0, es1, es2, es3, gs0, gs1, gs2, gs3, ss0, ss1, ss2, ss3,
          zsem):
        cid = lax.axis_index("c")
        sid = lax.axis_index("s")
        wid = cid * NS + sid
        ebufs = (e0, e1, e2, e3)
        esems = (es0, es1, es2, es3)
        rbufs = (r0, r1, r2, r3)
        gsems = (gs0, gs1, gs2, gs3)
        ssems = (ss0, ss1, ss2, ss3)

        z = jnp.zeros((16,), _f32)
        for j in range(16):
            for q in range(OUT // 16):
                zb_v[j, pl.ds(q * 16, 16)] = z

        def zacc(i, carry):
            pltpu.async_copy(zb_v, agg_sh.at[pl.ds(sid * 640 + i * 16, 16)],
                             zsem)
            return carry
        lax.fori_loop(0, 40, zacc, 0)

        def zaccw(i, carry):
            pltpu.make_async_copy(
                zb_v, agg_sh.at[pl.ds(sid * 640 + i * 16, 16)], zsem).wait()
            return carry
        lax.fori_loop(0, 40, zaccw, 0)
        plsc.subcore_barrier()

        def stage(ci, b4):
            pltpu.async_copy(cd_h.at[wid * CPW + ci], ebufs[b4], esems[b4])

        def wait_stage(ci, b4):
            pltpu.make_async_copy(cd_h.at[wid * CPW + ci], ebufs[b4],
                                  esems[b4]).wait()

        def gather(b4):
            pltpu.async_copy(h2_h.at[ebufs[b4].at[0]], rbufs[b4], gsems[b4])

        def wait_gather(b4):
            pltpu.make_async_copy(h2_h.at[ebufs[b4].at[0]], rbufs[b4],
                                  gsems[b4]).wait()

        def scatter(b4):
            pltpu.async_copy(rbufs[b4], agg_sh.at[ebufs[b4].at[1]],
                             ssems[b4], add=True)

        def wait_scatter(b4):
            pltpu.make_async_copy(rbufs[b4], agg_sh.at[ebufs[b4].at[1]],
                                  ssems[b4]).wait()

        stage(0, 0)
        stage(1, 1)
        wait_stage(0, 0)
        gather(0)

        def section(ci, b4):
            @pl.when(ci + 1 < CPW)
            def _():
                wait_stage(ci + 1, (b4 + 1) % 4)
                gather((b4 + 1) % 4)

            @pl.when(ci + 2 < CPW)
            def _():
                @pl.when(ci >= 2)
                def _():
                    wait_scatter((b4 + 2) % 4)
                stage(ci + 2, (b4 + 2) % 4)
            wait_gather(b4)
            scatter(b4)

        def quad(q4, carry):
            for s in range(4):
                section(q4 * 4 + s, s)
            return carry
        lax.fori_loop(0, CPW // 4, quad, 0)
        for b in range(4):
            wait_scatter(b)

        plsc.subcore_barrier()
        pltpu.sync_copy(agg_sh.at[pl.ds(sid * 640, 640)],
                        agg_out.at[cid, pl.ds(sid * 640, 640)])

    return k(cdata, h2)


_RB = 1000


def _tc_b(acc_pair, deg_pair, norm_degrees, W_ego, b_ego, W_gcn):
    def body(acc_ref, deg_ref, nd_ref, we_ref, be_ref, wg_ref, h2_ref):
        ego = jnp.concatenate([acc_ref[0], acc_ref[1]], axis=1) * nd_ref[...]
        h1 = jnp.maximum(
            jnp.dot(ego, we_ref[...], preferred_element_type=_f32)
            + be_ref[...], 0.0)
        deg = deg_ref[0] + deg_ref[1] + 1.0
        dis = lax.rsqrt(jnp.maximum(deg, 1e-12))
        h2_ref[...] = dis * jnp.dot(h1, wg_ref[...],
                                    preferred_element_type=_f32)

    return pl.pallas_call(
        body,
        grid=(N // _RB,),
        in_specs=[
            pl.BlockSpec((NC, _RB, DH), lambda i: (0, i, 0)),
            pl.BlockSpec((NC, _RB, 1), lambda i: (0, i, 0)),
            pl.BlockSpec((_RB, 1), lambda i: (i, 0)),
            pl.BlockSpec((D, D), lambda i: (0, 0)),
            pl.BlockSpec((1, D), lambda i: (0, 0)),
            pl.BlockSpec((D, OUT), lambda i: (0, 0)),
        ],
        out_specs=pl.BlockSpec((_RB, OUT), lambda i: (i, 0)),
        out_shape=jax.ShapeDtypeStruct((N, OUT), _f32),
    )(acc_pair, deg_pair, norm_degrees, W_ego, b_ego.reshape(1, D), W_gcn)


def _tc_d(agg_pair, h2, deg_pair, b_gcn):
    def body(agg_ref, h2_ref, deg_ref, bg_ref, out_ref):
        deg = deg_ref[0] + deg_ref[1] + 1.0
        dis = lax.rsqrt(jnp.maximum(deg, 1e-12))
        g = dis * (agg_ref[0] + agg_ref[1] + h2_ref[...]) + bg_ref[...]
        m = jnp.max(g, axis=1, keepdims=True)
        lse = m + jnp.log(jnp.sum(jnp.exp(g - m), axis=1, keepdims=True))
        out_ref[...] = g - lse

    return pl.pallas_call(
        body,
        grid=(N // _RB,),
        in_specs=[
            pl.BlockSpec((NC, _RB, OUT), lambda i: (0, i, 0)),
            pl.BlockSpec((_RB, OUT), lambda i: (i, 0)),
            pl.BlockSpec((NC, _RB, 1), lambda i: (0, i, 0)),
            pl.BlockSpec((1, OUT), lambda i: (0, 0)),
        ],
        out_specs=pl.BlockSpec((_RB, OUT), lambda i: (i, 0)),
        out_shape=jax.ShapeDtypeStruct((N, OUT), _f32),
    )(agg_pair, h2, deg_pair, b_gcn.reshape(1, OUT))


def kernel(x_in, edge_index_in, ego_ind, ego_val, norm_degrees, W_ego,
           b_ego, W_gcn, b_gcn):
    src = jnp.pad(ego_ind[:, 1, :].reshape(-1), (0, EP16 - E)).reshape(
        NS * CHT, CH)
    dst = jnp.pad(ego_ind[:, 0, :].reshape(-1), (0, EP16 - E)).reshape(
        NS * CHT, CH)
    val = lax.bitcast_convert_type(
        jnp.pad(ego_val.reshape(-1), (0, EP16 - E)), _i32).reshape(
        NS * CHT, CH)
    edata = jnp.stack([src, dst, val], axis=1)
    row = jnp.pad(edge_index_in[0], (0, EPC - E)).reshape(NW * CPW, CH)
    col = jnp.pad(edge_index_in[1], (0, EPC - E),
                  constant_values=N).reshape(NW * CPW, CH)
    cdata = jnp.stack([row, col], axis=1)
    x_halves = jnp.stack([x_in[:, :DH], x_in[:, DH:]])

    acc_full, deg_full = _sc_phase_a(edata, cdata, x_halves)
    deg_all = deg_full.reshape(NC, NPAD, 1)
    h2 = _tc_b(acc_full, deg_all, norm_degrees, W_ego, b_ego, W_gcn)
    agg_full = _sc_phase_c(cdata, h2)
    return _tc_d(agg_full, h2, deg_all, b_gcn)


# speedup vs baseline: 1.3820x; 1.0319x over previous
import functools

import jax
import jax.numpy as jnp
from jax import lax
from jax.experimental import pallas as pl
from jax.experimental.pallas import tpu as pltpu
from jax.experimental.pallas import tpu_sc as plsc

N = 10000
D = 128
OUT = 64
E = 320000
K_EGO = 4
E_EGO = 80000

NC = 2
NS = 16
NW = NC * NS
CH = 64
CHT = 316
EP16 = NS * CH * CHT
CPW = 160
EPC = NW * CH * CPW
DH = D // NC
NPAD = NW * 640

_f32 = jnp.float32
_i32 = jnp.int32

_SC_PARAMS = pltpu.CompilerParams(needs_layout_passes=False,
                                  use_tc_tiling_on_sc=False)


def _sc_phase_a(edata, cdata, x):
    mesh = plsc.VectorSubcoreMesh(core_axis_name="c", subcore_axis_name="s",
                                  num_cores=NC, num_subcores=NS)

    @functools.partial(
        pl.kernel,
        out_type=(
            jax.ShapeDtypeStruct((NC, NPAD, DH), _f32),
            jax.ShapeDtypeStruct((NC, NPAD), _f32),
        ),
        mesh=mesh,
        compiler_params=_SC_PARAMS,
        scratch_types=[
            pltpu.VMEM_SHARED((NPAD, DH), _f32),
            pltpu.VMEM_SHARED((NPAD,), _f32),
            pltpu.VMEM((3, CH), _i32),
            pltpu.VMEM((3, CH), _i32),
            pltpu.VMEM((3, CH), _i32),
            pltpu.VMEM((3, CH), _i32),
            pltpu.VMEM((CH, DH), _f32),
            pltpu.VMEM((CH, DH), _f32),
            pltpu.VMEM((CH, DH), _f32),
            pltpu.VMEM((CH, DH), _f32),
            pltpu.VMEM((CH,), _i32),
            pltpu.VMEM((CH,), _i32),
            pltpu.VMEM((CH,), _f32),
            pltpu.VMEM((16, DH), _f32),
            pltpu.SemaphoreType.DMA,
            pltpu.SemaphoreType.DMA,
            pltpu.SemaphoreType.DMA,
            pltpu.SemaphoreType.DMA,
            pltpu.SemaphoreType.DMA,
            pltpu.SemaphoreType.DMA,
            pltpu.SemaphoreType.DMA,
            pltpu.SemaphoreType.DMA,
            pltpu.SemaphoreType.DMA,
            pltpu.SemaphoreType.DMA,
            pltpu.SemaphoreType.DMA,
            pltpu.SemaphoreType.DMA,
            pltpu.SemaphoreType.DMA,
            pltpu.SemaphoreType.DMA,
            pltpu.SemaphoreType.DMA,
        ],
    )
    def k(ed_h, cd_h, x_h, acc_out, deg_out,
          acc_sh, deg_sh, e0, e1, e2, e3, r0, r1, r2, r3, c0, c1,
          ones_v, zb_v, es0, es1, es2, es3, gs0, gs1, gs2, gs3,
          ss0, ss1, ss2, ss3, cs0, cs1, zsem):
        cid = lax.axis_index("c")
        sid = lax.axis_index("s")
        wid = cid * NS + sid
        lane = lax.iota(_i32, 16)
        ebufs = (e0, e1, e2, e3)
        esems = (es0, es1, es2, es3)
        rbufs = (r0, r1, r2, r3)
        gsems = (gs0, gs1, gs2, gs3)
        ssems = (ss0, ss1, ss2, ss3)
        cbufs = (c0, c1)
        csems = (cs0, cs1)

        z = jnp.zeros((16,), _f32)
        one = jnp.ones((16,), _f32)
        for j in range(16):
            for q in range(DH // 16):
                zb_v[j, pl.ds(q * 16, 16)] = z

        for g in range(CH // 16):
            ones_v[pl.ds(g * 16, 16)] = z

        def zacc(i, carry):
            pltpu.async_copy(zb_v, acc_sh.at[pl.ds(sid * 640 + i * 16, 16)],
                             zsem)
            return carry
        lax.fori_loop(0, 40, zacc, 0)
        for j in range(10):
            pltpu.async_copy(ones_v,
                             deg_sh.at[pl.ds(sid * 640 + j * CH, CH)], zsem)

        def zaccw(i, carry):
            pltpu.make_async_copy(
                zb_v, acc_sh.at[pl.ds(sid * 640 + i * 16, 16)], zsem).wait()
            return carry
        lax.fori_loop(0, 40, zaccw, 0)
        for j in range(10):
            pltpu.make_async_copy(
                ones_v, deg_sh.at[pl.ds(sid * 640 + j * CH, CH)],
                zsem).wait()
        for g in range(CH // 16):
            ones_v[pl.ds(g * 16, 16)] = one
        plsc.subcore_barrier()

        def stage(ci, b4):
            pltpu.async_copy(ed_h.at[sid * CHT + ci], ebufs[b4], esems[b4])

        def wait_stage(ci, b4):
            pltpu.make_async_copy(ed_h.at[sid * CHT + ci], ebufs[b4],
                                  esems[b4]).wait()

        def gather(b4):
            pltpu.async_copy(x_h.at[cid].at[ebufs[b4].at[0]],
                             rbufs[b4], gsems[b4])

        def wait_gather(b4):
            pltpu.make_async_copy(x_h.at[cid].at[ebufs[b4].at[0]],
                                  rbufs[b4], gsems[b4]).wait()

        def scatter(b4):
            pltpu.async_copy(rbufs[b4], acc_sh.at[ebufs[b4].at[1]],
                             ssems[b4], add=True)

        def wait_scatter(b4):
            pltpu.make_async_copy(rbufs[b4], acc_sh.at[ebufs[b4].at[1]],
                                  ssems[b4]).wait()

        def scale(b4):
            rows = rbufs[b4]
            eb = ebufs[b4]
            two = jnp.full((16,), 2, _i32)

            def grp(g, carry):
                base = g * 16
                vals16 = plsc.bitcast(
                    plsc.load_gather(eb, [two, base + lane]), _f32)
                for e16 in range(16):
                    e = base + e16
                    eidx = jnp.full((16,), e, _i32)
                    vb = vals16.at[jnp.full((16,), e16, _i32)].get(
                        mode="promise_in_bounds")
                    for q in range(DH // 16):
                        lidx = q * 16 + lane
                        r = plsc.load_gather(rows, [eidx, lidx])
                        plsc.store_scatter(rows, [eidx, lidx], r * vb)
                return carry
            lax.fori_loop(0, CH // 16, grp, 0)

        stage(0, 0)
        stage(1, 1)
        wait_stage(0, 0)
        gather(0)

        def section(ci, b4):
            @pl.when(ci + 1 < CHT)
            def _():
                wait_stage(ci + 1, (b4 + 1) % 4)
                gather((b4 + 1) % 4)

            @pl.when(ci + 2 < CHT)
            def _():
                @pl.when(ci >= 2)
                def _():
                    wait_scatter((b4 + 2) % 4)
                stage(ci + 2, (b4 + 2) % 4)
            wait_gather(b4)
            scale(b4)
            scatter(b4)

        def quad(q4, carry):
            for s in range(4):
                section(q4 * 4 + s, s)
            return carry
        lax.fori_loop(0, CHT // 4, quad, 0)
        for b in range(4):
            wait_scatter(b)

        pltpu.async_copy(cd_h.at[wid * CPW, 1], c0, cs0)

        def dchunk(ci2, carry):
            for b in range(2):
                ci = ci2 * 2 + b

                @pl.when(ci + 1 < CPW)
                def _():
                    pltpu.async_copy(cd_h.at[wid * CPW + ci + 1, 1],
                                     cbufs[1 - b], csems[1 - b])
                pltpu.make_async_copy(cd_h.at[wid * CPW + ci, 1],
                                      cbufs[b], csems[b]).wait()
                pltpu.sync_copy(ones_v, deg_sh.at[cbufs[b]], add=True)
            return carry
        lax.fori_loop(0, CPW // 2, dchunk, 0)

        plsc.subcore_barrier()

        pltpu.sync_copy(acc_sh.at[pl.ds(sid * 640, 640)],
                        acc_out.at[cid, pl.ds(sid * 640, 640)])
        pltpu.sync_copy(deg_sh.at[pl.ds(sid * 640, 640)],
                        deg_out.at[cid, pl.ds(sid * 640, 640)])

    return k(edata, cdata, x)


def _sc_phase_c(cdata, h2):
    mesh = plsc.VectorSubcoreMesh(core_axis_name="c", subcore_axis_name="s",
                                  num_cores=NC, num_subcores=NS)

    @functools.partial(
        pl.kernel,
        out_type=jax.ShapeDtypeStruct((NC, NPAD, OUT), _f32),
        mesh=mesh,
        compiler_params=_SC_PARAMS,
        scratch_types=[
            pltpu.VMEM_SHARED((NPAD, OUT), _f32),
            pltpu.VMEM((2, CH), _i32),
            pltpu.VMEM((2, CH), _i32),
            pltpu.VMEM((2, CH), _i32),
            pltpu.VMEM((2, CH), _i32),
            pltpu.VMEM((CH, OUT), _f32),
            pltpu.VMEM((CH, OUT), _f32),
            pltpu.VMEM((CH, OUT), _f32),
            pltpu.VMEM((CH, OUT), _f32),
            pltpu.VMEM((16, OUT), _f32),
            pltpu.SemaphoreType.DMA,
            pltpu.SemaphoreType.DMA,
            pltpu.SemaphoreType.DMA,
            pltpu.SemaphoreType.DMA,
            pltpu.SemaphoreType.DMA,
            pltpu.SemaphoreType.DMA,
            pltpu.SemaphoreType.DMA,
            pltpu.SemaphoreType.DMA,
            pltpu.SemaphoreType.DMA,
            pltpu.SemaphoreType.DMA,
            pltpu.SemaphoreType.DMA,
            pltpu.SemaphoreType.DMA,
            pltpu.SemaphoreType.DMA,
        ],
    )
    def k(cd_h, h2_h, agg_out, agg_sh, e0, e1, e2, e3, r0, r1, r2, r3,
          zb_v, es0, es1, es2, es3, gs0, gs1, gs2, gs3, ss0, ss1, ss2, ss3,
          zsem):
        cid = lax.axis_index("c")
        sid = lax.axis_index("s")
        wid = sid * NC + cid
        ebufs = (e0, e1, e2, e3)
        esems = (es0, es1, es2, es3)
        rbufs = (r0, r1, r2, r3)
        gsems = (gs0, gs1, gs2, gs3)
        ssems = (ss0, ss1, ss2, ss3)

        z = jnp.zeros((16,), _f32)
        for j in range(16):
            for q in range(OUT // 16):
                zb_v[j, pl.ds(q * 16, 16)] = z

        def zacc(i, carry):
            pltpu.async_copy(zb_v, agg_sh.at[pl.ds(sid * 640 + i * 16, 16)],
                             zsem)
            return carry
        lax.fori_loop(0, 40, zacc, 0)

        def zaccw(i, carry):
            pltpu.make_async_copy(
                zb_v, agg_sh.at[pl.ds(sid * 640 + i * 16, 16)], zsem).wait()
            return carry
        lax.fori_loop(0, 40, zaccw, 0)
        plsc.subcore_barrier()

        def stage(ci, b4):
            pltpu.async_copy(cd_h.at[wid * CPW + ci], ebufs[b4], esems[b4])

        def wait_stage(ci, b4):
            pltpu.make_async_copy(cd_h.at[wid * CPW + ci], ebufs[b4],
                                  esems[b4]).wait()

        def gather(b4):
            pltpu.async_copy(h2_h.at[ebufs[b4].at[0]], rbufs[b4], gsems[b4])

        def wait_gather(b4):
            pltpu.make_async_copy(h2_h.at[ebufs[b4].at[0]], rbufs[b4],
                                  gsems[b4]).wait()

        def scatter(b4):
            pltpu.async_copy(rbufs[b4], agg_sh.at[ebufs[b4].at[1]],
                             ssems[b4], add=True)

        def wait_scatter(b4):
            pltpu.make_async_copy(rbufs[b4], agg_sh.at[ebufs[b4].at[1]],
                                  ssems[b4]).wait()

        stage(0, 0)
        stage(1, 1)
        wait_stage(0, 0)
        gather(0)

        def section(ci, b4):
            @pl.when(ci + 1 < CPW)
            def _():
                wait_stage(ci + 1, (b4 + 1) % 4)
                gather((b4 + 1) % 4)

            @pl.when(ci + 2 < CPW)
            def _():
                @pl.when(ci >= 2)
                def _():
                    wait_scatter((b4 + 2) % 4)
                stage(ci + 2, (b4 + 2) % 4)
            wait_gather(b4)
            scatter(b4)

        def quad(q4, carry):
            for s in range(4):
                section(q4 * 4 + s, s)
            return carry
        lax.fori_loop(0, CPW // 4, quad, 0)
        for b in range(4):
            wait_scatter(b)

        plsc.subcore_barrier()
        pltpu.sync_copy(agg_sh.at[pl.ds(sid * 640, 640)],
                        agg_out.at[cid, pl.ds(sid * 640, 640)])

    return k(cdata, h2)


_RB = 1000


def _tc_b(acc_pair, deg_pair, norm_degrees, W_ego, b_ego, W_gcn):
    def body(acc_ref, deg_ref, nd_ref, we_ref, be_ref, wg_ref, h2_ref):
        ego = jnp.concatenate([acc_ref[0], acc_ref[1]], axis=1) * nd_ref[...]
        h1 = jnp.maximum(
            jnp.dot(ego, we_ref[...], preferred_element_type=_f32)
            + be_ref[...], 0.0)
        deg = deg_ref[0] + deg_ref[1] + 1.0
        dis = lax.rsqrt(jnp.maximum(deg, 1e-12))
        h2_ref[...] = dis * jnp.dot(h1, wg_ref[...],
                                    preferred_element_type=_f32)

    return pl.pallas_call(
        body,
        grid=(N // _RB,),
        in_specs=[
            pl.BlockSpec((NC, _RB, DH), lambda i: (0, i, 0)),
            pl.BlockSpec((NC, _RB, 1), lambda i: (0, i, 0)),
            pl.BlockSpec((_RB, 1), lambda i: (i, 0)),
            pl.BlockSpec((D, D), lambda i: (0, 0)),
            pl.BlockSpec((1, D), lambda i: (0, 0)),
            pl.BlockSpec((D, OUT), lambda i: (0, 0)),
        ],
        out_specs=pl.BlockSpec((_RB, OUT), lambda i: (i, 0)),
        out_shape=jax.ShapeDtypeStruct((N, OUT), _f32),
    )(acc_pair, deg_pair, norm_degrees, W_ego, b_ego.reshape(1, D), W_gcn)


def _tc_d(agg_pair, h2, deg_pair, b_gcn):
    def body(agg_ref, h2_ref, deg_ref, bg_ref, out_ref):
        deg = deg_ref[0] + deg_ref[1] + 1.0
        dis = lax.rsqrt(jnp.maximum(deg, 1e-12))
        g = dis * (agg_ref[0] + agg_ref[1] + h2_ref[...]) + bg_ref[...]
        m = jnp.max(g, axis=1, keepdims=True)
        lse = m + jnp.log(jnp.sum(jnp.exp(g - m), axis=1, keepdims=True))
        out_ref[...] = g - lse

    return pl.pallas_call(
        body,
        grid=(N // _RB,),
        in_specs=[
            pl.BlockSpec((NC, _RB, OUT), lambda i: (0, i, 0)),
            pl.BlockSpec((_RB, OUT), lambda i: (i, 0)),
            pl.BlockSpec((NC, _RB, 1), lambda i: (0, i, 0)),
            pl.BlockSpec((1, OUT), lambda i: (0, 0)),
        ],
        out_specs=pl.BlockSpec((_RB, OUT), lambda i: (i, 0)),
        out_shape=jax.ShapeDtypeStruct((N, OUT), _f32),
    )(agg_pair, h2, deg_pair, b_gcn.reshape(1, OUT))


def kernel(x_in, edge_index_in, ego_ind, ego_val, norm_degrees, W_ego,
           b_ego, W_gcn, b_gcn):
    src = jnp.pad(ego_ind[:, 1, :].reshape(-1), (0, EP16 - E)).reshape(
        NS * CHT, CH)
    dst = jnp.pad(ego_ind[:, 0, :].reshape(-1), (0, EP16 - E)).reshape(
        NS * CHT, CH)
    val = lax.bitcast_convert_type(
        jnp.pad(ego_val.reshape(-1), (0, EP16 - E)), _i32).reshape(
        NS * CHT, CH)
    edata = jnp.stack([src, dst, val], axis=1)
    row = jnp.pad(edge_index_in[0], (0, EPC - E)).reshape(NW * CPW, CH)
    col = jnp.pad(edge_index_in[1], (0, EPC - E),
                  constant_values=N).reshape(NW * CPW, CH)
    cdata = jnp.stack([row, col], axis=1)
    x_halves = jnp.stack([x_in[:, :DH], x_in[:, DH:]])

    acc_full, deg_full = _sc_phase_a(edata, cdata, x_halves)
    deg_all = deg_full.reshape(NC, NPAD, 1)
    h2 = _tc_b(acc_full, deg_all, norm_degrees, W_ego, b_ego, W_gcn)
    agg_full = _sc_phase_c(cdata, h2)
    return _tc_d(agg_full, h2, deg_all, b_gcn)
